# baseline (speedup 1.0000x reference)
import functools

import jax
import jax.numpy as jnp
from jax import lax
from jax.experimental import pallas as pl
from jax.experimental.pallas import tpu as pltpu
from jax.experimental.pallas import tpu_sc as plsc

_L = 16
_S = 4096
_ROW = 50
_M = 1000000

_mesh = plsc.VectorSubcoreMesh(core_axis_name="c", subcore_axis_name="s")


def _log_f32(x):
    xi = plsc.bitcast(x, jnp.int32)
    e = (xi >> 23) - 127
    m = plsc.bitcast((xi & 0x7FFFFF) | (127 << 23), jnp.float32)
    t = (m - 1.0) / (m + 1.0)
    t2 = t * t
    poly = 1.0 + t2 * (1.0 / 3.0 + t2 * (1.0 / 5.0 + t2 * (1.0 / 7.0 + t2 / 9.0)))
    ln_m = 2.0 * t * poly
    return e.astype(jnp.float32) * 0.6931471805599453 + ln_m


@functools.partial(
    pl.kernel,
    out_type=jax.ShapeDtypeStruct((_L,), jnp.int32),
    mesh=_mesh,
    compiler_params=pltpu.CompilerParams(needs_layout_passes=False),
    scratch_types=[
        pltpu.VMEM((_L,), jnp.int32),
        pltpu.VMEM((1, _ROW), jnp.int32),
        pltpu.VMEM((_L,), jnp.int32),
        pltpu.VMEM((_S,), jnp.float32),
        pltpu.VMEM((_L,), jnp.float32),
        pltpu.VMEM((_L,), jnp.int32),
        pltpu.VMEM((_L,), jnp.float32),
    ],
)
def _sc_add(size_hbm, nsl_hbm, nseq_hbm, nlsd_hbm,
            seq_ref, len_ref, lp_ref, out_size_hbm,
            vs, vrow, vn, vx, vlp, vwl, vwp):
    c = lax.axis_index("c")
    s = lax.axis_index("s")

    @pl.when(jnp.logical_and(c == 0, s == 0))
    def _():
        pltpu.sync_copy(size_hbm, vs)
        pltpu.sync_copy(nsl_hbm, vn)
        pltpu.sync_copy(nseq_hbm, vrow)
        pltpu.sync_copy(nlsd_hbm, vx)

        idx = lax.reduce_max(vs[...], axes=(0,))

        pltpu.sync_copy(vrow, seq_ref.at[pl.ds(idx, 1), :])

        base = pl.multiple_of(jnp.minimum((idx >> 3) << 3, _M - _L), 8)
        off = idx - base
        lane = lax.iota(jnp.int32, _L)
        hit = lane == off

        pltpu.sync_copy(len_ref.at[pl.ds(base, _L)], vwl)
        vwl[...] = jnp.where(hit, vn[...], vwl[...])
        pltpu.sync_copy(vwl, len_ref.at[pl.ds(base, _L)])

        def max_body(i, acc):
            return jnp.maximum(acc, vx[pl.ds(i * _L, _L)])

        mvec = lax.fori_loop(1, _S // _L, max_body, vx[pl.ds(0, _L)])
        mmax = jnp.full((_L,), jnp.max(mvec))

        def sum_body(i, acc):
            return acc + jnp.exp(vx[pl.ds(i * _L, _L)] - mmax)

        svec = lax.fori_loop(0, _S // _L, sum_body, jnp.zeros((_L,), jnp.float32))
        tot = jnp.full((_L,), jnp.sum(svec))
        vlp[...] = mmax + _log_f32(tot)

        pltpu.sync_copy(lp_ref.at[pl.ds(base, _L)], vwp)
        vwp[...] = jnp.where(hit, vlp[...], vwp[...])
        pltpu.sync_copy(vwp, lp_ref.at[pl.ds(base, _L)])

        vs[...] = vs[...] + 1
        pltpu.sync_copy(vs, out_size_hbm)


def kernel(sequences, sequence_lengths, log_probabilities, size,
           node_sequence, node_sequence_length, node_log_state_distribution):
    size16 = jnp.broadcast_to(jnp.asarray(size, jnp.int32), (_L,))
    nsl16 = jnp.broadcast_to(jnp.asarray(node_sequence_length, jnp.int32), (_L,))
    nseq2d = jnp.asarray(node_sequence, jnp.int32).reshape(1, _ROW)

    seq_ref = jax.new_ref(sequences)
    len_ref = jax.new_ref(sequence_lengths)
    lp_ref = jax.new_ref(log_probabilities)

    out16 = _sc_add(size16, nsl16, nseq2d, node_log_state_distribution,
                    seq_ref, len_ref, lp_ref)

    return seq_ref[...], len_ref[...], lp_ref[...], out16[0]


# speedup vs baseline: 4.0549x; 4.0549x over previous
import functools

import jax
import jax.numpy as jnp
from jax import lax
from jax.experimental import pallas as pl
from jax.experimental.pallas import tpu as pltpu
from jax.experimental.pallas import tpu_sc as plsc

_L = 16
_S = 4096
_ROW = 50
_M = 1000000

_mesh = plsc.VectorSubcoreMesh(core_axis_name="c", subcore_axis_name="s")


def _log_f32(x):
    xi = plsc.bitcast(x, jnp.int32)
    e = (xi >> 23) - 127
    m = plsc.bitcast((xi & 0x7FFFFF) | (127 << 23), jnp.float32)
    t = (m - 1.0) / (m + 1.0)
    t2 = t * t
    poly = 1.0 + t2 * (1.0 / 3.0 + t2 * (1.0 / 5.0 + t2 * (1.0 / 7.0 + t2 / 9.0)))
    ln_m = 2.0 * t * poly
    return e.astype(jnp.float32) * 0.6931471805599453 + ln_m


@functools.partial(
    pl.kernel,
    out_type=jax.ShapeDtypeStruct((_L,), jnp.int32),
    mesh=_mesh,
    compiler_params=pltpu.CompilerParams(needs_layout_passes=False),
    scratch_types=[
        pltpu.VMEM((_L,), jnp.int32),
        pltpu.VMEM((_ROW, _L), jnp.int32),
        pltpu.VMEM((_L,), jnp.int32),
        pltpu.VMEM((_S,), jnp.float32),
        pltpu.VMEM((_L,), jnp.float32),
        pltpu.VMEM((_L,), jnp.int32),
        pltpu.VMEM((_L,), jnp.float32),
        pltpu.VMEM((_ROW, 128), jnp.int32),
    ],
)
def _sc_add(size_hbm, nsl_hbm, nseq_hbm, nlsd_hbm,
            seq_ref, len_ref, lp_ref, out_size_hbm,
            vs, vrow, vn, vx, vlp, vwl, vwp, vw):
    c = lax.axis_index("c")
    s = lax.axis_index("s")

    @pl.when(jnp.logical_and(c == 0, s == 0))
    def _():
        pltpu.sync_copy(size_hbm, vs)
        pltpu.sync_copy(nsl_hbm, vn)
        pltpu.sync_copy(nseq_hbm, vrow)
        pltpu.sync_copy(nlsd_hbm, vx)

        idx = lax.reduce_max(vs[...], axes=(0,))

        base = pl.multiple_of(jnp.minimum((idx >> 3) << 3, _M - _L), 8)
        off = idx - base
        lane = lax.iota(jnp.int32, _L)
        hit = lane == off

        base128 = pl.multiple_of((idx >> 7) << 7, 128)
        off128 = idx - base128
        chunk = pl.multiple_of((off128 >> 4) << 4, 16)
        hit16 = (lane + chunk) == off128
        pltpu.sync_copy(seq_ref.at[:, pl.ds(base128, 128)], vw)
        for j in range(_ROW):
            vw[j, pl.ds(chunk, _L)] = jnp.where(
                hit16, vrow[j, :], vw[j, pl.ds(chunk, _L)])
        pltpu.sync_copy(vw, seq_ref.at[:, pl.ds(base128, 128)])

        pltpu.sync_copy(len_ref.at[pl.ds(base, _L)], vwl)
        vwl[...] = jnp.where(hit, vn[...], vwl[...])
        pltpu.sync_copy(vwl, len_ref.at[pl.ds(base, _L)])

        def max_body(i, acc):
            return jnp.maximum(acc, vx[pl.ds(i * _L, _L)])

        mvec = lax.fori_loop(1, _S // _L, max_body, vx[pl.ds(0, _L)])
        mmax = jnp.full((_L,), jnp.max(mvec))

        def sum_body(i, acc):
            return acc + jnp.exp(vx[pl.ds(i * _L, _L)] - mmax)

        svec = lax.fori_loop(0, _S // _L, sum_body, jnp.zeros((_L,), jnp.float32))
        tot = jnp.full((_L,), jnp.sum(svec))
        vlp[...] = mmax + _log_f32(tot)

        pltpu.sync_copy(lp_ref.at[pl.ds(base, _L)], vwp)
        vwp[...] = jnp.where(hit, vlp[...], vwp[...])
        pltpu.sync_copy(vwp, lp_ref.at[pl.ds(base, _L)])

        vs[...] = vs[...] + 1
        pltpu.sync_copy(vs, out_size_hbm)


def kernel(sequences, sequence_lengths, log_probabilities, size,
           node_sequence, node_sequence_length, node_log_state_distribution):
    size16 = jnp.broadcast_to(jnp.asarray(size, jnp.int32), (_L,))
    nsl16 = jnp.broadcast_to(jnp.asarray(node_sequence_length, jnp.int32), (_L,))
    nseq_b = jnp.broadcast_to(
        jnp.asarray(node_sequence, jnp.int32)[:, None], (_ROW, _L))

    seq_ref = jax.new_ref(sequences.T)
    len_ref = jax.new_ref(sequence_lengths)
    lp_ref = jax.new_ref(log_probabilities)

    out16 = _sc_add(size16, nsl16, nseq_b, node_log_state_distribution,
                    seq_ref, len_ref, lp_ref)

    return seq_ref[...].T, len_ref[...], lp_ref[...], out16[0]
